# Initial kernel scaffold; baseline (speedup 1.0000x reference)
import jax
import jax.numpy as jnp
from jax.experimental import pallas as pl


def kernel(x, edge_index, W_struct, W_task):
    raise NotImplementedError("write your pallas kernel here")


# speedup vs baseline: 1.6194x; 1.6194x over previous
import functools

import jax
import jax.numpy as jnp
from jax import lax
from jax.experimental import pallas as pl
from jax.experimental.pallas import tpu as pltpu

N = 10000
E = 160000
D = 256
C = 64
K = 16
DH = D // 2

_BN = 400
_GRID = N // _BN


def _dense_body(x_ref, sL_ref, sR_ref, cnt_ref, wsLT_ref, wsRT_ref,
                wxT_ref, wnLT_ref, wnRT_ref, out_ref):
    cnt = cnt_ref[:, 0:1]
    denom = cnt + 1.0
    xb = x_ref[...]
    xLb = xb[:, :DH]
    xRb = xb[:, DH:]
    nL = (sL_ref[...] + xLb) / denom
    nR = (sR_ref[...] + xRb) / denom

    logits = jnp.dot(nL, wsLT_ref[...]) + jnp.dot(nR, wsRT_ref[...])
    rowmax = jnp.max(logits, axis=1, keepdims=True)
    iota_c = lax.broadcasted_iota(jnp.int32, logits.shape, 1)
    idx = jnp.min(jnp.where(logits == rowmax, iota_c, C), axis=1, keepdims=True)

    scores = (jnp.dot(xb, wxT_ref[...])
              + jnp.dot(nL, wnLT_ref[...])
              + jnp.dot(nR, wnRT_ref[...]))
    col_cluster = lax.broadcasted_iota(jnp.int32, scores.shape, 1) // K
    masked = jnp.where(col_cluster == idx, scores, 0.0)
    w = C * K
    while w > K:
        w //= 2
        masked = masked[:, :w] + masked[:, w:2 * w]
    out_ref[...] = masked


def _dense_stage(x, sumL, sumR, cnt16, W_struct, W_task):
    W_flat = W_task.reshape(C * K, 2 * D)
    wxT = W_flat[:, :D].T
    wnLT = W_flat[:, D:D + DH].T
    wnRT = W_flat[:, D + DH:].T
    wsLT = W_struct[:, :DH].T
    wsRT = W_struct[:, DH:].T

    row = lambda i: (i, 0)
    rep = lambda i: (0, 0)
    return pl.pallas_call(
        _dense_body,
        grid=(_GRID,),
        in_specs=[
            pl.BlockSpec((_BN, D), row),
            pl.BlockSpec((_BN, DH), row),
            pl.BlockSpec((_BN, DH), row),
            pl.BlockSpec((_BN, 16), row),
            pl.BlockSpec((DH, C), rep),
            pl.BlockSpec((DH, C), rep),
            pl.BlockSpec((D, C * K), rep),
            pl.BlockSpec((DH, C * K), rep),
            pl.BlockSpec((DH, C * K), rep),
        ],
        out_specs=pl.BlockSpec((_BN, K), row),
        out_shape=jax.ShapeDtypeStruct((N, K), jnp.float32),
    )(x, sumL, sumR, cnt16, wsLT, wsRT, wxT, wnLT, wnRT)


def kernel(x, edge_index, W_struct, W_task):
    src = edge_index[0]
    dst = edge_index[1]
    msg = jnp.take(x, src, axis=0)
    summed = jax.ops.segment_sum(msg, dst, num_segments=N)
    cnt = jax.ops.segment_sum(jnp.ones((E,), jnp.float32), dst, num_segments=N)
    sumL = summed[:, :DH]
    sumR = summed[:, DH:]
    cnt16 = jnp.broadcast_to(cnt[:, None], (N, 16))
    return _dense_stage(x, sumL, sumR, cnt16, W_struct, W_task)


# speedup vs baseline: 3.9472x; 2.4375x over previous
import functools

import jax
import jax.numpy as jnp
from jax import lax
from jax.experimental import pallas as pl
from jax.experimental.pallas import tpu as pltpu
from jax.experimental.pallas import tpu_sc as plsc

N = 10000
E = 160000
D = 256
C = 64
K = 16
DH = D // 2

_BN = 400
_GRID = N // _BN

_NT = 16
_EB = 64
_GC = 16
_NGROUP = 10
_CNT_SPLIT = _NGROUP // 2
_NCHUNK = _GC * _NGROUP
_EPT = _NCHUNK * _EB
_ACC_ROWS = N + _NT
_ZSTRIPE = _ACC_ROWS // _NT
_HROWS = 80
_WSTRIPE = 624
_WLAST = N - 15 * _WSTRIPE


def _sc_body(xL_h, xR_h, src_h, dst_h, hi_h, lo_h, eye_h, zrow_h,
             sumL_o, sumR_o, cntA_o, cntB_o,
             src_v, dst_v, hi_v, lo_v, rowbuf, acc_sh, hist_sh, sem):
    c = lax.axis_index("c")
    s = lax.axis_index("s")

    pltpu.sync_copy(zrow_h, acc_sh.at[pl.ds(s * _ZSTRIPE, _ZSTRIPE)])

    @pl.when(s < _HROWS // 8)
    def _():
        pltpu.sync_copy(zrow_h.at[pl.ds(0, 8)], hist_sh.at[pl.ds(s * 8, 8)])

    plsc.subcore_barrier()

    def edge_loop(xsrc):
        def group(g, carry):
            pltpu.sync_copy(src_h.at[s, pl.ds(g * _GC, _GC)], src_v)
            pltpu.sync_copy(dst_h.at[s, pl.ds(g * _GC, _GC)], dst_v)
            pltpu.sync_copy(hi_h.at[s, pl.ds(g * _GC, _GC)], hi_v)
            pltpu.sync_copy(lo_h.at[s, pl.ds(g * _GC, _GC)], lo_v)
            count_here = jnp.where(c == 0, g < _CNT_SPLIT, g >= _CNT_SPLIT)

            def body(j, carry2):
                pltpu.async_copy(xsrc.at[src_v.at[j]], rowbuf, sem).wait()
                pltpu.sync_copy(rowbuf, acc_sh.at[dst_v.at[j]], add=True)

                @pl.when(count_here)
                def _():
                    pltpu.async_copy(eye_h.at[lo_v.at[j]], rowbuf, sem).wait()
                    pltpu.sync_copy(rowbuf, hist_sh.at[hi_v.at[j]], add=True)
                return carry2
            lax.fori_loop(0, _GC, body, 0)
            return carry
        lax.fori_loop(0, _NGROUP, group, 0)

    @pl.when(c == 0)
    def _():
        edge_loop(xL_h)

    @pl.when(c == 1)
    def _():
        edge_loop(xR_h)

    plsc.subcore_barrier()

    def writeback(nrows):
        rows = pl.ds(s * _WSTRIPE, nrows)

        @pl.when(c == 0)
        def _():
            pltpu.sync_copy(acc_sh.at[rows], sumL_o.at[rows])

        @pl.when(c == 1)
        def _():
            pltpu.sync_copy(acc_sh.at[rows], sumR_o.at[rows])

    @pl.when(s < _NT - 1)
    def _():
        writeback(_WSTRIPE)

    @pl.when(s == _NT - 1)
    def _():
        writeback(_WLAST)

    @pl.when(s < _HROWS // 8)
    def _():
        hrows = pl.ds(s * 8, 8)

        @pl.when(c == 0)
        def _():
            pltpu.sync_copy(hist_sh.at[hrows], cntA_o.at[hrows])

        @pl.when(c == 1)
        def _():
            pltpu.sync_copy(hist_sh.at[hrows], cntB_o.at[hrows])


def _sc_stage(x, edge_index):
    src = edge_index[0].reshape(_NT, N)
    dst = edge_index[1].reshape(_NT, N)
    pad = _EPT - N
    srcp = jnp.concatenate(
        [src, jnp.zeros((_NT, pad), jnp.int32)], axis=1).reshape(_NT, _NCHUNK, _EB)
    dstf = jnp.concatenate(
        [dst, jnp.full((_NT, pad), N, jnp.int32)], axis=1)
    dstp = dstf.reshape(_NT, _NCHUNK, _EB)
    hip = (dstf >> 7).reshape(_NT, _NCHUNK, _EB)
    lop = (dstf & 127).reshape(_NT, _NCHUNK, _EB)
    xL = x[:, :DH]
    xR = x[:, DH:]
    eye = jnp.eye(DH, dtype=jnp.float32)
    zrow = jnp.zeros((_ZSTRIPE, DH), jnp.float32)

    mesh = plsc.VectorSubcoreMesh(core_axis_name="c", subcore_axis_name="s")
    f = functools.partial(
        pl.kernel, mesh=mesh,
        out_type=[
            jax.ShapeDtypeStruct((N, DH), jnp.float32),
            jax.ShapeDtypeStruct((N, DH), jnp.float32),
            jax.ShapeDtypeStruct((_HROWS, DH), jnp.float32),
            jax.ShapeDtypeStruct((_HROWS, DH), jnp.float32),
        ],
        scratch_types=[
            pltpu.VMEM((_GC, _EB), jnp.int32),
            pltpu.VMEM((_GC, _EB), jnp.int32),
            pltpu.VMEM((_GC, _EB), jnp.int32),
            pltpu.VMEM((_GC, _EB), jnp.int32),
            pltpu.VMEM((_EB, DH), jnp.float32),
            pltpu.VMEM_SHARED((_ACC_ROWS, DH), jnp.float32),
            pltpu.VMEM_SHARED((_HROWS, DH), jnp.float32),
            pltpu.SemaphoreType.DMA,
        ],
    )(_sc_body)
    return f(xL, xR, srcp, dstp, hip, lop, eye, zrow)


def _dense_body(x_ref, sL_ref, sR_ref, cnt_ref, wsLT_ref, wsRT_ref,
                wxT_ref, wnLT_ref, wnRT_ref, out_ref):
    cnt = cnt_ref[:, 0:1]
    denom = cnt + 1.0
    xb = x_ref[...]
    xLb = xb[:, :DH]
    xRb = xb[:, DH:]
    nL = (sL_ref[...] + xLb) / denom
    nR = (sR_ref[...] + xRb) / denom

    logits = jnp.dot(nL, wsLT_ref[...]) + jnp.dot(nR, wsRT_ref[...])
    rowmax = jnp.max(logits, axis=1, keepdims=True)
    iota_c = lax.broadcasted_iota(jnp.int32, logits.shape, 1)
    idx = jnp.min(jnp.where(logits == rowmax, iota_c, C), axis=1, keepdims=True)

    scores = (jnp.dot(xb, wxT_ref[...])
              + jnp.dot(nL, wnLT_ref[...])
              + jnp.dot(nR, wnRT_ref[...]))
    col_cluster = lax.broadcasted_iota(jnp.int32, scores.shape, 1) // K
    masked = jnp.where(col_cluster == idx, scores, 0.0)
    w = C * K
    while w > K:
        w //= 2
        masked = masked[:, :w] + masked[:, w:2 * w]
    out_ref[...] = masked


def _dense_stage(x, sumL, sumR, cnt16, W_struct, W_task):
    W_flat = W_task.reshape(C * K, 2 * D)
    wxT = W_flat[:, :D].T
    wnLT = W_flat[:, D:D + DH].T
    wnRT = W_flat[:, D + DH:].T
    wsLT = W_struct[:, :DH].T
    wsRT = W_struct[:, DH:].T

    row = lambda i: (i, 0)
    rep = lambda i: (0, 0)
    return pl.pallas_call(
        _dense_body,
        grid=(_GRID,),
        in_specs=[
            pl.BlockSpec((_BN, D), row),
            pl.BlockSpec((_BN, DH), row),
            pl.BlockSpec((_BN, DH), row),
            pl.BlockSpec((_BN, 16), row),
            pl.BlockSpec((DH, C), rep),
            pl.BlockSpec((DH, C), rep),
            pl.BlockSpec((D, C * K), rep),
            pl.BlockSpec((DH, C * K), rep),
            pl.BlockSpec((DH, C * K), rep),
        ],
        out_specs=pl.BlockSpec((_BN, K), row),
        out_shape=jax.ShapeDtypeStruct((N, K), jnp.float32),
    )(x, sumL, sumR, cnt16, wsLT, wsRT, wxT, wnLT, wnRT)


def kernel(x, edge_index, W_struct, W_task):
    sumL, sumR, cntA, cntB = _sc_stage(x, edge_index)
    cnt = (cntA + cntB).reshape(_HROWS * DH)[:N]
    cnt16 = jnp.broadcast_to(cnt[:, None], (N, 16))
    return _dense_stage(x, sumL, sumR, cnt16, W_struct, W_task)


# speedup vs baseline: 4.8828x; 1.2370x over previous
import functools

import jax
import jax.numpy as jnp
from jax import lax
from jax.experimental import pallas as pl
from jax.experimental.pallas import tpu as pltpu
from jax.experimental.pallas import tpu_sc as plsc

N = 10000
E = 160000
D = 256
C = 64
K = 16
DH = D // 2

_BN = 400
_GRID = N // _BN

_NT = 16
_EB = 64
_GC = 16
_NGROUP = 10
_CNT_SPLIT = _NGROUP // 2
_NCHUNK = _GC * _NGROUP
_EPT = _NCHUNK * _EB
_ACC_ROWS = N + _NT
_ZSTRIPE = _ACC_ROWS // _NT
_HROWS = 80
_WSTRIPE = 624
_WLAST = N - 15 * _WSTRIPE


def _sc_body(xL_h, xR_h, src_h, dst_h, hi_h, lo_h, eye_h, zrow_h,
             sumL_o, sumR_o, cntA_o, cntB_o,
             src_v, dst_v, hi_v, lo_v, xb0, xb1, ob0, ob1,
             acc_sh, hist_sh, sg0, sg1, so0, so1):
    c = lax.axis_index("c")
    s = lax.axis_index("s")

    pltpu.sync_copy(zrow_h, acc_sh.at[pl.ds(s * _ZSTRIPE, _ZSTRIPE)])

    @pl.when(s < _HROWS // 8)
    def _():
        pltpu.sync_copy(zrow_h.at[pl.ds(0, 8)], hist_sh.at[pl.ds(s * 8, 8)])

    plsc.subcore_barrier()

    def edge_loop(xsrc):
        def group(g, carry):
            pltpu.sync_copy(src_h.at[s, pl.ds(g * _GC, _GC)], src_v)
            pltpu.sync_copy(dst_h.at[s, pl.ds(g * _GC, _GC)], dst_v)
            pltpu.sync_copy(hi_h.at[s, pl.ds(g * _GC, _GC)], hi_v)
            pltpu.sync_copy(lo_h.at[s, pl.ds(g * _GC, _GC)], lo_v)
            count_here = jnp.where(c == 0, g < _CNT_SPLIT, g >= _CNT_SPLIT)

            def body(m, carry2):
                j0 = 2 * m
                j1 = 2 * m + 1
                h0 = pltpu.async_copy(xsrc.at[src_v.at[j0]], xb0, sg0)
                h1 = pltpu.async_copy(xsrc.at[src_v.at[j1]], xb1, sg1)

                @pl.when(count_here)
                def _():
                    pltpu.async_copy(eye_h.at[lo_v.at[j0]], ob0, so0)
                    pltpu.async_copy(eye_h.at[lo_v.at[j1]], ob1, so1)

                h0.wait()
                pltpu.sync_copy(xb0, acc_sh.at[dst_v.at[j0]], add=True)
                h1.wait()
                pltpu.sync_copy(xb1, acc_sh.at[dst_v.at[j1]], add=True)

                @pl.when(count_here)
                def _():
                    pltpu.make_async_copy(eye_h.at[pl.ds(0, _EB)], ob0, so0).wait()
                    pltpu.sync_copy(ob0, hist_sh.at[hi_v.at[j0]], add=True)
                    pltpu.make_async_copy(eye_h.at[pl.ds(0, _EB)], ob1, so1).wait()
                    pltpu.sync_copy(ob1, hist_sh.at[hi_v.at[j1]], add=True)
                return carry2
            lax.fori_loop(0, _GC // 2, body, 0)
            return carry
        lax.fori_loop(0, _NGROUP, group, 0)

    @pl.when(c == 0)
    def _():
        edge_loop(xL_h)

    @pl.when(c == 1)
    def _():
        edge_loop(xR_h)

    plsc.subcore_barrier()

    def writeback(nrows):
        rows = pl.ds(s * _WSTRIPE, nrows)

        @pl.when(c == 0)
        def _():
            pltpu.sync_copy(acc_sh.at[rows], sumL_o.at[rows])

        @pl.when(c == 1)
        def _():
            pltpu.sync_copy(acc_sh.at[rows], sumR_o.at[rows])

    @pl.when(s < _NT - 1)
    def _():
        writeback(_WSTRIPE)

    @pl.when(s == _NT - 1)
    def _():
        writeback(_WLAST)

    @pl.when(s < _HROWS // 8)
    def _():
        hrows = pl.ds(s * 8, 8)

        @pl.when(c == 0)
        def _():
            pltpu.sync_copy(hist_sh.at[hrows], cntA_o.at[hrows])

        @pl.when(c == 1)
        def _():
            pltpu.sync_copy(hist_sh.at[hrows], cntB_o.at[hrows])


def _sc_stage(x, edge_index):
    src = edge_index[0].reshape(_NT, N)
    dst = edge_index[1].reshape(_NT, N)
    pad = _EPT - N
    srcp = jnp.concatenate(
        [src, jnp.zeros((_NT, pad), jnp.int32)], axis=1).reshape(_NT, _NCHUNK, _EB)
    dstf = jnp.concatenate(
        [dst, jnp.full((_NT, pad), N, jnp.int32)], axis=1)
    dstp = dstf.reshape(_NT, _NCHUNK, _EB)
    hip = (dstf >> 7).reshape(_NT, _NCHUNK, _EB)
    lop = (dstf & 127).reshape(_NT, _NCHUNK, _EB)
    xL = x[:, :DH]
    xR = x[:, DH:]
    eye = jnp.eye(DH, dtype=jnp.float32)
    zrow = jnp.zeros((_ZSTRIPE, DH), jnp.float32)

    mesh = plsc.VectorSubcoreMesh(core_axis_name="c", subcore_axis_name="s")
    f = functools.partial(
        pl.kernel, mesh=mesh,
        out_type=[
            jax.ShapeDtypeStruct((N, DH), jnp.float32),
            jax.ShapeDtypeStruct((N, DH), jnp.float32),
            jax.ShapeDtypeStruct((_HROWS, DH), jnp.float32),
            jax.ShapeDtypeStruct((_HROWS, DH), jnp.float32),
        ],
        scratch_types=[
            pltpu.VMEM((_GC, _EB), jnp.int32),
            pltpu.VMEM((_GC, _EB), jnp.int32),
            pltpu.VMEM((_GC, _EB), jnp.int32),
            pltpu.VMEM((_GC, _EB), jnp.int32),
            pltpu.VMEM((_EB, DH), jnp.float32),
            pltpu.VMEM((_EB, DH), jnp.float32),
            pltpu.VMEM((_EB, DH), jnp.float32),
            pltpu.VMEM((_EB, DH), jnp.float32),
            pltpu.VMEM_SHARED((_ACC_ROWS, DH), jnp.float32),
            pltpu.VMEM_SHARED((_HROWS, DH), jnp.float32),
            pltpu.SemaphoreType.DMA,
            pltpu.SemaphoreType.DMA,
            pltpu.SemaphoreType.DMA,
            pltpu.SemaphoreType.DMA,
        ],
    )(_sc_body)
    return f(xL, xR, srcp, dstp, hip, lop, eye, zrow)


def _dense_body(x_ref, sL_ref, sR_ref, cnt_ref, wsLT_ref, wsRT_ref,
                wxT_ref, wnLT_ref, wnRT_ref, out_ref):
    cnt = cnt_ref[:, 0:1]
    denom = cnt + 1.0
    xb = x_ref[...]
    xLb = xb[:, :DH]
    xRb = xb[:, DH:]
    nL = (sL_ref[...] + xLb) / denom
    nR = (sR_ref[...] + xRb) / denom

    logits = jnp.dot(nL, wsLT_ref[...]) + jnp.dot(nR, wsRT_ref[...])
    rowmax = jnp.max(logits, axis=1, keepdims=True)
    iota_c = lax.broadcasted_iota(jnp.int32, logits.shape, 1)
    idx = jnp.min(jnp.where(logits == rowmax, iota_c, C), axis=1, keepdims=True)

    scores = (jnp.dot(xb, wxT_ref[...])
              + jnp.dot(nL, wnLT_ref[...])
              + jnp.dot(nR, wnRT_ref[...]))
    col_cluster = lax.broadcasted_iota(jnp.int32, scores.shape, 1) // K
    masked = jnp.where(col_cluster == idx, scores, 0.0)
    w = C * K
    while w > K:
        w //= 2
        masked = masked[:, :w] + masked[:, w:2 * w]
    out_ref[...] = masked


def _dense_stage(x, sumL, sumR, cnt16, W_struct, W_task):
    W_flat = W_task.reshape(C * K, 2 * D)
    wxT = W_flat[:, :D].T
    wnLT = W_flat[:, D:D + DH].T
    wnRT = W_flat[:, D + DH:].T
    wsLT = W_struct[:, :DH].T
    wsRT = W_struct[:, DH:].T

    row = lambda i: (i, 0)
    rep = lambda i: (0, 0)
    return pl.pallas_call(
        _dense_body,
        grid=(_GRID,),
        in_specs=[
            pl.BlockSpec((_BN, D), row),
            pl.BlockSpec((_BN, DH), row),
            pl.BlockSpec((_BN, DH), row),
            pl.BlockSpec((_BN, 16), row),
            pl.BlockSpec((DH, C), rep),
            pl.BlockSpec((DH, C), rep),
            pl.BlockSpec((D, C * K), rep),
            pl.BlockSpec((DH, C * K), rep),
            pl.BlockSpec((DH, C * K), rep),
        ],
        out_specs=pl.BlockSpec((_BN, K), row),
        out_shape=jax.ShapeDtypeStruct((N, K), jnp.float32),
    )(x, sumL, sumR, cnt16, wsLT, wsRT, wxT, wnLT, wnRT)


def kernel(x, edge_index, W_struct, W_task):
    sumL, sumR, cntA, cntB = _sc_stage(x, edge_index)
    cnt = (cntA + cntB).reshape(_HROWS * DH)[:N]
    cnt16 = jnp.broadcast_to(cnt[:, None], (N, 16))
    return _dense_stage(x, sumL, sumR, cnt16, W_struct, W_task)


# speedup vs baseline: 4.9681x; 1.0175x over previous
import functools

import jax
import jax.numpy as jnp
from jax import lax
from jax.experimental import pallas as pl
from jax.experimental.pallas import tpu as pltpu
from jax.experimental.pallas import tpu_sc as plsc

N = 10000
E = 160000
D = 256
C = 64
K = 16
DH = D // 2

_BN = 400
_GRID = N // _BN

_NT = 16
_EB = 64
_GC = 16
_NGROUP = 10
_CNT_SPLIT = _NGROUP // 2
_NCHUNK = _GC * _NGROUP
_EPT = _NCHUNK * _EB
_ACC_ROWS = N + _NT
_ZSTRIPE = _ACC_ROWS // _NT
_HROWS = 80
_WSTRIPE = 624
_WLAST = N - 15 * _WSTRIPE


def _sc_body(xL_h, xR_h, src_h, dst_h, hi_h, lo_h, eye_h, zrow_h,
             sumL_o, sumR_o, cntA_o, cntB_o,
             src_v, dst_v, hi_v, lo_v, xb0, xb1, ob0, ob1,
             acc_sh, hist_sh, sg0, sg1, so0, so1, ss0, ss1, ss2, ss3):
    c = lax.axis_index("c")
    s = lax.axis_index("s")

    pltpu.sync_copy(zrow_h, acc_sh.at[pl.ds(s * _ZSTRIPE, _ZSTRIPE)])

    @pl.when(s < _HROWS // 8)
    def _():
        pltpu.sync_copy(zrow_h.at[pl.ds(0, 8)], hist_sh.at[pl.ds(s * 8, 8)])

    plsc.subcore_barrier()

    def edge_loop(xsrc):
        def group(g, carry):
            pltpu.sync_copy(src_h.at[s, pl.ds(g * _GC, _GC)], src_v)
            pltpu.sync_copy(dst_h.at[s, pl.ds(g * _GC, _GC)], dst_v)
            pltpu.sync_copy(hi_h.at[s, pl.ds(g * _GC, _GC)], hi_v)
            pltpu.sync_copy(lo_h.at[s, pl.ds(g * _GC, _GC)], lo_v)
            count_here = jnp.where(c == 0, g < _CNT_SPLIT, g >= _CNT_SPLIT)

            def body(m, carry2):
                j0 = 2 * m
                j1 = 2 * m + 1
                h0 = pltpu.async_copy(xsrc.at[src_v.at[j0]], xb0, sg0)
                h1 = pltpu.async_copy(xsrc.at[src_v.at[j1]], xb1, sg1)

                @pl.when(count_here)
                def _():
                    pltpu.async_copy(eye_h.at[lo_v.at[j0]], ob0, so0)
                    pltpu.async_copy(eye_h.at[lo_v.at[j1]], ob1, so1)

                h0.wait()
                w0 = pltpu.async_copy(xb0, acc_sh.at[dst_v.at[j0]], ss0, add=True)
                h1.wait()
                w1 = pltpu.async_copy(xb1, acc_sh.at[dst_v.at[j1]], ss1, add=True)

                @pl.when(count_here)
                def _():
                    pltpu.make_async_copy(eye_h.at[pl.ds(0, _EB)], ob0, so0).wait()
                    pltpu.async_copy(ob0, hist_sh.at[hi_v.at[j0]], ss2, add=True)
                    pltpu.make_async_copy(eye_h.at[pl.ds(0, _EB)], ob1, so1).wait()
                    pltpu.async_copy(ob1, hist_sh.at[hi_v.at[j1]], ss3, add=True)

                w0.wait()
                w1.wait()

                @pl.when(count_here)
                def _():
                    pltpu.make_async_copy(ob0, hist_sh.at[pl.ds(0, _EB)], ss2).wait()
                    pltpu.make_async_copy(ob1, hist_sh.at[pl.ds(0, _EB)], ss3).wait()
                return carry2
            lax.fori_loop(0, _GC // 2, body, 0)
            return carry
        lax.fori_loop(0, _NGROUP, group, 0)

    @pl.when(c == 0)
    def _():
        edge_loop(xL_h)

    @pl.when(c == 1)
    def _():
        edge_loop(xR_h)

    plsc.subcore_barrier()

    def writeback(nrows):
        rows = pl.ds(s * _WSTRIPE, nrows)

        @pl.when(c == 0)
        def _():
            pltpu.sync_copy(acc_sh.at[rows], sumL_o.at[rows])

        @pl.when(c == 1)
        def _():
            pltpu.sync_copy(acc_sh.at[rows], sumR_o.at[rows])

    @pl.when(s < _NT - 1)
    def _():
        writeback(_WSTRIPE)

    @pl.when(s == _NT - 1)
    def _():
        writeback(_WLAST)

    @pl.when(s < _HROWS // 8)
    def _():
        hrows = pl.ds(s * 8, 8)

        @pl.when(c == 0)
        def _():
            pltpu.sync_copy(hist_sh.at[hrows], cntA_o.at[hrows])

        @pl.when(c == 1)
        def _():
            pltpu.sync_copy(hist_sh.at[hrows], cntB_o.at[hrows])


def _sc_stage(x, edge_index):
    src = edge_index[0].reshape(_NT, N)
    dst = edge_index[1].reshape(_NT, N)
    pad = _EPT - N
    srcp = jnp.concatenate(
        [src, jnp.zeros((_NT, pad), jnp.int32)], axis=1).reshape(_NT, _NCHUNK, _EB)
    dstf = jnp.concatenate(
        [dst, jnp.full((_NT, pad), N, jnp.int32)], axis=1)
    dstp = dstf.reshape(_NT, _NCHUNK, _EB)
    hip = (dstf >> 7).reshape(_NT, _NCHUNK, _EB)
    lop = (dstf & 127).reshape(_NT, _NCHUNK, _EB)
    xL = x[:, :DH]
    xR = x[:, DH:]
    eye = jnp.eye(DH, dtype=jnp.float32)
    zrow = jnp.zeros((_ZSTRIPE, DH), jnp.float32)

    mesh = plsc.VectorSubcoreMesh(core_axis_name="c", subcore_axis_name="s")
    f = functools.partial(
        pl.kernel, mesh=mesh,
        out_type=[
            jax.ShapeDtypeStruct((N, DH), jnp.float32),
            jax.ShapeDtypeStruct((N, DH), jnp.float32),
            jax.ShapeDtypeStruct((_HROWS, DH), jnp.float32),
            jax.ShapeDtypeStruct((_HROWS, DH), jnp.float32),
        ],
        scratch_types=[
            pltpu.VMEM((_GC, _EB), jnp.int32),
            pltpu.VMEM((_GC, _EB), jnp.int32),
            pltpu.VMEM((_GC, _EB), jnp.int32),
            pltpu.VMEM((_GC, _EB), jnp.int32),
            pltpu.VMEM((_EB, DH), jnp.float32),
            pltpu.VMEM((_EB, DH), jnp.float32),
            pltpu.VMEM((_EB, DH), jnp.float32),
            pltpu.VMEM((_EB, DH), jnp.float32),
            pltpu.VMEM_SHARED((_ACC_ROWS, DH), jnp.float32),
            pltpu.VMEM_SHARED((_HROWS, DH), jnp.float32),
            pltpu.SemaphoreType.DMA,
            pltpu.SemaphoreType.DMA,
            pltpu.SemaphoreType.DMA,
            pltpu.SemaphoreType.DMA,
            pltpu.SemaphoreType.DMA,
            pltpu.SemaphoreType.DMA,
            pltpu.SemaphoreType.DMA,
            pltpu.SemaphoreType.DMA,
        ],
    )(_sc_body)
    return f(xL, xR, srcp, dstp, hip, lop, eye, zrow)


def _dense_body(x_ref, sL_ref, sR_ref, cnt_ref, wsLT_ref, wsRT_ref,
                wxT_ref, wnLT_ref, wnRT_ref, out_ref):
    cnt = cnt_ref[:, 0:1]
    denom = cnt + 1.0
    xb = x_ref[...]
    xLb = xb[:, :DH]
    xRb = xb[:, DH:]
    nL = (sL_ref[...] + xLb) / denom
    nR = (sR_ref[...] + xRb) / denom

    logits = jnp.dot(nL, wsLT_ref[...]) + jnp.dot(nR, wsRT_ref[...])
    rowmax = jnp.max(logits, axis=1, keepdims=True)
    iota_c = lax.broadcasted_iota(jnp.int32, logits.shape, 1)
    idx = jnp.min(jnp.where(logits == rowmax, iota_c, C), axis=1, keepdims=True)

    scores = (jnp.dot(xb, wxT_ref[...])
              + jnp.dot(nL, wnLT_ref[...])
              + jnp.dot(nR, wnRT_ref[...]))
    col_cluster = lax.broadcasted_iota(jnp.int32, scores.shape, 1) // K
    masked = jnp.where(col_cluster == idx, scores, 0.0)
    w = C * K
    while w > K:
        w //= 2
        masked = masked[:, :w] + masked[:, w:2 * w]
    out_ref[...] = masked


def _dense_stage(x, sumL, sumR, cnt16, W_struct, W_task):
    W_flat = W_task.reshape(C * K, 2 * D)
    wxT = W_flat[:, :D].T
    wnLT = W_flat[:, D:D + DH].T
    wnRT = W_flat[:, D + DH:].T
    wsLT = W_struct[:, :DH].T
    wsRT = W_struct[:, DH:].T

    row = lambda i: (i, 0)
    rep = lambda i: (0, 0)
    return pl.pallas_call(
        _dense_body,
        grid=(_GRID,),
        in_specs=[
            pl.BlockSpec((_BN, D), row),
            pl.BlockSpec((_BN, DH), row),
            pl.BlockSpec((_BN, DH), row),
            pl.BlockSpec((_BN, 16), row),
            pl.BlockSpec((DH, C), rep),
            pl.BlockSpec((DH, C), rep),
            pl.BlockSpec((D, C * K), rep),
            pl.BlockSpec((DH, C * K), rep),
            pl.BlockSpec((DH, C * K), rep),
        ],
        out_specs=pl.BlockSpec((_BN, K), row),
        out_shape=jax.ShapeDtypeStruct((N, K), jnp.float32),
    )(x, sumL, sumR, cnt16, wsLT, wsRT, wxT, wnLT, wnRT)


def kernel(x, edge_index, W_struct, W_task):
    sumL, sumR, cntA, cntB = _sc_stage(x, edge_index)
    cnt = (cntA + cntB).reshape(_HROWS * DH)[:N]
    cnt16 = jnp.broadcast_to(cnt[:, None], (N, 16))
    return _dense_stage(x, sumL, sumR, cnt16, W_struct, W_task)


# speedup vs baseline: 6.2109x; 1.2502x over previous
import functools

import jax
import jax.numpy as jnp
from jax import lax
from jax.experimental import pallas as pl
from jax.experimental.pallas import tpu as pltpu
from jax.experimental.pallas import tpu_sc as plsc

N = 10000
E = 160000
D = 256
C = 64
K = 16
DH = D // 2

_BN = 400
_GRID = N // _BN

_NT = 16
_EB = 128
_GC = 8
_NGROUP = 10
_NCHUNK = _GC * _NGROUP
_EPT = _NCHUNK * _EB
_ACC_ROWS = N + _NT
_ZSTRIPE = _ACC_ROWS // _NT
_HROWS = 80
_WSTRIPE = 624
_WLAST = N - 15 * _WSTRIPE


def _sc_body(xL_h, xR_h, src_h, dst_h, zrow_h,
             sumL_o, sumR_o,
             src_v, dst_v, xb0, xb1,
             acc_sh, sg0, sg1, ss0, ss1):
    c = lax.axis_index("c")
    s = lax.axis_index("s")

    pltpu.sync_copy(zrow_h, acc_sh.at[pl.ds(s * _ZSTRIPE, _ZSTRIPE)])
    plsc.subcore_barrier()

    def edge_loop(xsrc):
        def group(g, carry):
            pltpu.sync_copy(src_h.at[s, pl.ds(g * _GC, _GC)], src_v)
            pltpu.sync_copy(dst_h.at[s, pl.ds(g * _GC, _GC)], dst_v)

            def body(m, carry2):
                j0 = 2 * m
                j1 = 2 * m + 1
                h0 = pltpu.async_copy(xsrc.at[src_v.at[j0]], xb0, sg0)
                h1 = pltpu.async_copy(xsrc.at[src_v.at[j1]], xb1, sg1)
                h0.wait()
                w0 = pltpu.async_copy(xb0, acc_sh.at[dst_v.at[j0]], ss0, add=True)
                h1.wait()
                w1 = pltpu.async_copy(xb1, acc_sh.at[dst_v.at[j1]], ss1, add=True)
                w0.wait()
                w1.wait()
                return carry2
            lax.fori_loop(0, _GC // 2, body, 0)
            return carry
        lax.fori_loop(0, _NGROUP, group, 0)

    @pl.when(c == 0)
    def _():
        edge_loop(xL_h)

    @pl.when(c == 1)
    def _():
        edge_loop(xR_h)

    plsc.subcore_barrier()

    def writeback(nrows):
        rows = pl.ds(s * _WSTRIPE, nrows)

        @pl.when(c == 0)
        def _():
            pltpu.sync_copy(acc_sh.at[rows], sumL_o.at[rows])

        @pl.when(c == 1)
        def _():
            pltpu.sync_copy(acc_sh.at[rows], sumR_o.at[rows])

    @pl.when(s < _NT - 1)
    def _():
        writeback(_WSTRIPE)

    @pl.when(s == _NT - 1)
    def _():
        writeback(_WLAST)


def _sc_stage(x, edge_index):
    src = edge_index[0].reshape(_NT, N)
    dst = edge_index[1].reshape(_NT, N)
    pad = _EPT - N
    srcp = jnp.concatenate(
        [src, jnp.zeros((_NT, pad), jnp.int32)], axis=1).reshape(_NT, _NCHUNK, _EB)
    dstp = jnp.concatenate(
        [dst, jnp.full((_NT, pad), N, jnp.int32)], axis=1).reshape(_NT, _NCHUNK, _EB)
    xL = x[:, :DH]
    xR = x[:, DH:]
    zrow = jnp.zeros((_ZSTRIPE, DH), jnp.float32)

    mesh = plsc.VectorSubcoreMesh(core_axis_name="c", subcore_axis_name="s")
    f = functools.partial(
        pl.kernel, mesh=mesh,
        out_type=[
            jax.ShapeDtypeStruct((N, DH), jnp.float32),
            jax.ShapeDtypeStruct((N, DH), jnp.float32),
        ],
        scratch_types=[
            pltpu.VMEM((_GC, _EB), jnp.int32),
            pltpu.VMEM((_GC, _EB), jnp.int32),
            pltpu.VMEM((_EB, DH), jnp.float32),
            pltpu.VMEM((_EB, DH), jnp.float32),
            pltpu.VMEM_SHARED((_ACC_ROWS, DH), jnp.float32),
            pltpu.SemaphoreType.DMA,
            pltpu.SemaphoreType.DMA,
            pltpu.SemaphoreType.DMA,
            pltpu.SemaphoreType.DMA,
        ],
    )(_sc_body)
    return f(xL, xR, srcp, dstp, zrow)


_CEB = 2000
_CGRID = E // _CEB


def _cnt_body(dst_ref, out_ref):
    i = pl.program_id(0)

    @pl.when(i == 0)
    def _():
        out_ref[...] = jnp.zeros_like(out_ref)

    d = dst_ref[...].reshape(1, _CEB)
    hi = d >> 7
    lo = d & 127
    oh_hi = (lax.broadcasted_iota(jnp.int32, (_HROWS, _CEB), 0)
             == hi).astype(jnp.float32)
    oh_lo = (lo.reshape(_CEB, 1)
             == lax.broadcasted_iota(jnp.int32, (_CEB, DH), 1)
             ).astype(jnp.float32)
    out_ref[...] += jnp.dot(oh_hi, oh_lo)


def _cnt_stage(edge_index):
    dstb = edge_index[1].reshape(_CGRID, 1, _CEB)
    return pl.pallas_call(
        _cnt_body,
        grid=(_CGRID,),
        in_specs=[pl.BlockSpec((1, 1, _CEB), lambda i: (i, 0, 0))],
        out_specs=pl.BlockSpec((_HROWS, DH), lambda i: (0, 0)),
        out_shape=jax.ShapeDtypeStruct((_HROWS, DH), jnp.float32),
    )(dstb)


def _dense_body(x_ref, sL_ref, sR_ref, cnt_ref, wsLT_ref, wsRT_ref,
                wxT_ref, wnLT_ref, wnRT_ref, out_ref):
    cnt = cnt_ref[:, 0:1]
    denom = cnt + 1.0
    xb = x_ref[...]
    xLb = xb[:, :DH]
    xRb = xb[:, DH:]
    nL = (sL_ref[...] + xLb) / denom
    nR = (sR_ref[...] + xRb) / denom

    logits = jnp.dot(nL, wsLT_ref[...]) + jnp.dot(nR, wsRT_ref[...])
    rowmax = jnp.max(logits, axis=1, keepdims=True)
    iota_c = lax.broadcasted_iota(jnp.int32, logits.shape, 1)
    idx = jnp.min(jnp.where(logits == rowmax, iota_c, C), axis=1, keepdims=True)

    scores = (jnp.dot(xb, wxT_ref[...])
              + jnp.dot(nL, wnLT_ref[...])
              + jnp.dot(nR, wnRT_ref[...]))
    col_cluster = lax.broadcasted_iota(jnp.int32, scores.shape, 1) // K
    masked = jnp.where(col_cluster == idx, scores, 0.0)
    w = C * K
    while w > K:
        w //= 2
        masked = masked[:, :w] + masked[:, w:2 * w]
    out_ref[...] = masked


def _dense_stage(x, sumL, sumR, cnt16, W_struct, W_task):
    W_flat = W_task.reshape(C * K, 2 * D)
    wxT = W_flat[:, :D].T
    wnLT = W_flat[:, D:D + DH].T
    wnRT = W_flat[:, D + DH:].T
    wsLT = W_struct[:, :DH].T
    wsRT = W_struct[:, DH:].T

    row = lambda i: (i, 0)
    rep = lambda i: (0, 0)
    return pl.pallas_call(
        _dense_body,
        grid=(_GRID,),
        in_specs=[
            pl.BlockSpec((_BN, D), row),
            pl.BlockSpec((_BN, DH), row),
            pl.BlockSpec((_BN, DH), row),
            pl.BlockSpec((_BN, 16), row),
            pl.BlockSpec((DH, C), rep),
            pl.BlockSpec((DH, C), rep),
            pl.BlockSpec((D, C * K), rep),
            pl.BlockSpec((DH, C * K), rep),
            pl.BlockSpec((DH, C * K), rep),
        ],
        out_specs=pl.BlockSpec((_BN, K), row),
        out_shape=jax.ShapeDtypeStruct((N, K), jnp.float32),
    )(x, sumL, sumR, cnt16, wsLT, wsRT, wxT, wnLT, wnRT)


def kernel(x, edge_index, W_struct, W_task):
    sumL, sumR = _sc_stage(x, edge_index)
    cnt2d = _cnt_stage(edge_index)
    cnt = cnt2d.reshape(_HROWS * DH)[:N]
    cnt16 = jnp.broadcast_to(cnt[:, None], (N, 16))
    return _dense_stage(x, sumL, sumR, cnt16, W_struct, W_task)


# speedup vs baseline: 6.8730x; 1.1066x over previous
import functools

import jax
import jax.numpy as jnp
from jax import lax
from jax.experimental import pallas as pl
from jax.experimental.pallas import tpu as pltpu
from jax.experimental.pallas import tpu_sc as plsc

N = 10000
E = 160000
D = 256
C = 64
K = 16
DH = D // 2

_BN = 400
_GRID = N // _BN

_NT = 16
_EB = 128
_GC = 40
_NGROUP = 2
_NCHUNK = _GC * _NGROUP
_EPT = _NCHUNK * _EB
_ACC_ROWS = N + _NT
_ZSTRIPE = _ACC_ROWS // _NT
_HROWS = 80
_WSTRIPE = 624
_WLAST = N - 15 * _WSTRIPE


def _sc_body(x_h, src_h, dst_h, zrow_h,
             sumL_o, sumR_o,
             src_v, dst_v, xb0, xb1,
             acc_sh, sg0, sg1, ss0, ss1):
    c = lax.axis_index("c")
    s = lax.axis_index("s")

    pltpu.sync_copy(zrow_h, acc_sh.at[pl.ds(s * _ZSTRIPE, _ZSTRIPE)])
    plsc.subcore_barrier()

    def edge_loop(xsrc):
        def group(g, carry):
            pltpu.sync_copy(src_h.at[s, pl.ds(g * _GC, _GC)], src_v)
            pltpu.sync_copy(dst_h.at[s, pl.ds(g * _GC, _GC)], dst_v)

            def body(m, carry2):
                j0 = 2 * m
                j1 = 2 * m + 1

                @pl.when(m > 0)
                def _():
                    pltpu.make_async_copy(
                        xb0, acc_sh.at[pl.ds(0, _EB)], ss0).wait()

                h0 = pltpu.async_copy(xsrc.at[src_v.at[j0]], xb0, sg0)

                @pl.when(m > 0)
                def _():
                    pltpu.make_async_copy(
                        xb1, acc_sh.at[pl.ds(0, _EB)], ss1).wait()

                h1 = pltpu.async_copy(xsrc.at[src_v.at[j1]], xb1, sg1)
                h0.wait()
                pltpu.async_copy(xb0, acc_sh.at[dst_v.at[j0]], ss0, add=True)
                h1.wait()
                pltpu.async_copy(xb1, acc_sh.at[dst_v.at[j1]], ss1, add=True)
                return carry2
            lax.fori_loop(0, _GC // 2, body, 0)
            pltpu.make_async_copy(xb0, acc_sh.at[pl.ds(0, _EB)], ss0).wait()
            pltpu.make_async_copy(xb1, acc_sh.at[pl.ds(0, _EB)], ss1).wait()
            return carry
        lax.fori_loop(0, _NGROUP, group, 0)

    @pl.when(c == 0)
    def _():
        edge_loop(x_h.at[:, pl.ds(0, DH)])

    @pl.when(c == 1)
    def _():
        edge_loop(x_h.at[:, pl.ds(DH, DH)])

    plsc.subcore_barrier()

    def writeback(nrows):
        rows = pl.ds(s * _WSTRIPE, nrows)

        @pl.when(c == 0)
        def _():
            pltpu.sync_copy(acc_sh.at[rows], sumL_o.at[rows])

        @pl.when(c == 1)
        def _():
            pltpu.sync_copy(acc_sh.at[rows], sumR_o.at[rows])

    @pl.when(s < _NT - 1)
    def _():
        writeback(_WSTRIPE)

    @pl.when(s == _NT - 1)
    def _():
        writeback(_WLAST)


def _sc_stage(x, edge_index):
    src = edge_index[0].reshape(_NT, N)
    dst = edge_index[1].reshape(_NT, N)
    pad = _EPT - N
    srcp = jnp.concatenate(
        [src, jnp.zeros((_NT, pad), jnp.int32)], axis=1).reshape(_NT, _NCHUNK, _EB)
    dstp = jnp.concatenate(
        [dst, jnp.full((_NT, pad), N, jnp.int32)], axis=1).reshape(_NT, _NCHUNK, _EB)
    zrow = jnp.zeros((_ZSTRIPE, DH), jnp.float32)

    mesh = plsc.VectorSubcoreMesh(core_axis_name="c", subcore_axis_name="s")
    f = functools.partial(
        pl.kernel, mesh=mesh,
        out_type=[
            jax.ShapeDtypeStruct((N, DH), jnp.float32),
            jax.ShapeDtypeStruct((N, DH), jnp.float32),
        ],
        scratch_types=[
            pltpu.VMEM((_GC, _EB), jnp.int32),
            pltpu.VMEM((_GC, _EB), jnp.int32),
            pltpu.VMEM((_EB, DH), jnp.float32),
            pltpu.VMEM((_EB, DH), jnp.float32),
            pltpu.VMEM_SHARED((_ACC_ROWS, DH), jnp.float32),
            pltpu.SemaphoreType.DMA,
            pltpu.SemaphoreType.DMA,
            pltpu.SemaphoreType.DMA,
            pltpu.SemaphoreType.DMA,
        ],
    )(_sc_body)
    return f(x, srcp, dstp, zrow)


_CEB = 2000
_CGRID = E // _CEB


def _cnt_body(dst_ref, out_ref):
    i = pl.program_id(0)

    @pl.when(i == 0)
    def _():
        out_ref[...] = jnp.zeros_like(out_ref)

    d = dst_ref[...].reshape(1, _CEB)
    hi = d >> 7
    lo = d & 127
    oh_hi = (lax.broadcasted_iota(jnp.int32, (_HROWS, _CEB), 0)
             == hi).astype(jnp.float32)
    oh_lo = (lo.reshape(_CEB, 1)
             == lax.broadcasted_iota(jnp.int32, (_CEB, DH), 1)
             ).astype(jnp.float32)
    out_ref[...] += jnp.dot(oh_hi, oh_lo)


def _cnt_stage(edge_index):
    dstb = edge_index[1].reshape(_CGRID, 1, _CEB)
    return pl.pallas_call(
        _cnt_body,
        grid=(_CGRID,),
        in_specs=[pl.BlockSpec((1, 1, _CEB), lambda i: (i, 0, 0))],
        out_specs=pl.BlockSpec((_HROWS, DH), lambda i: (0, 0)),
        out_shape=jax.ShapeDtypeStruct((_HROWS, DH), jnp.float32),
    )(dstb)


def _dense_body(x_ref, sL_ref, sR_ref, cnt_ref, wsLT_ref, wsRT_ref,
                wxT_ref, wnLT_ref, wnRT_ref, out_ref):
    cnt = cnt_ref[:, 0:1]
    denom = cnt + 1.0
    xb = x_ref[...]
    xLb = xb[:, :DH]
    xRb = xb[:, DH:]
    nL = (sL_ref[...] + xLb) / denom
    nR = (sR_ref[...] + xRb) / denom

    logits = jnp.dot(nL, wsLT_ref[...]) + jnp.dot(nR, wsRT_ref[...])
    rowmax = jnp.max(logits, axis=1, keepdims=True)
    iota_c = lax.broadcasted_iota(jnp.int32, logits.shape, 1)
    idx = jnp.min(jnp.where(logits == rowmax, iota_c, C), axis=1, keepdims=True)

    scores = (jnp.dot(xb, wxT_ref[...])
              + jnp.dot(nL, wnLT_ref[...])
              + jnp.dot(nR, wnRT_ref[...]))
    col_cluster = lax.broadcasted_iota(jnp.int32, scores.shape, 1) // K
    masked = jnp.where(col_cluster == idx, scores, 0.0)
    w = C * K
    while w > K:
        w //= 2
        masked = masked[:, :w] + masked[:, w:2 * w]
    out_ref[...] = masked


def _dense_stage(x, sumL, sumR, cnt16, W_struct, W_task):
    W_flat = W_task.reshape(C * K, 2 * D)
    wxT = W_flat[:, :D].T
    wnLT = W_flat[:, D:D + DH].T
    wnRT = W_flat[:, D + DH:].T
    wsLT = W_struct[:, :DH].T
    wsRT = W_struct[:, DH:].T

    row = lambda i: (i, 0)
    rep = lambda i: (0, 0)
    return pl.pallas_call(
        _dense_body,
        grid=(_GRID,),
        in_specs=[
            pl.BlockSpec((_BN, D), row),
            pl.BlockSpec((_BN, DH), row),
            pl.BlockSpec((_BN, DH), row),
            pl.BlockSpec((_BN, 16), row),
            pl.BlockSpec((DH, C), rep),
            pl.BlockSpec((DH, C), rep),
            pl.BlockSpec((D, C * K), rep),
            pl.BlockSpec((DH, C * K), rep),
            pl.BlockSpec((DH, C * K), rep),
        ],
        out_specs=pl.BlockSpec((_BN, K), row),
        out_shape=jax.ShapeDtypeStruct((N, K), jnp.float32),
    )(x, sumL, sumR, cnt16, wsLT, wsRT, wxT, wnLT, wnRT)


def kernel(x, edge_index, W_struct, W_task):
    sumL, sumR = _sc_stage(x, edge_index)
    cnt2d = _cnt_stage(edge_index)
    cnt = cnt2d.reshape(_HROWS * DH)[:N]
    cnt16 = jnp.broadcast_to(cnt[:, None], (N, 16))
    return _dense_stage(x, sumL, sumR, cnt16, W_struct, W_task)
